# baseline (speedup 1.0000x reference)
import functools

import jax
import jax.numpy as jnp
from jax import lax
from jax.experimental import pallas as pl
from jax.experimental.pallas import tpu as pltpu
from jax.experimental.pallas import tpu_sc as plsc

N = 10000
NFEAT = 128
NHID = 128
NCLASS = 64
DH = 64

_INFO = plsc.get_sparse_core_info()
NC = _INFO.num_cores
NS = _INFO.num_subcores
NW = NC * NS
BATCH = 128
ZROWS = 632
ACC_ROWS = NS * ZROWS
CP_CHUNK = 640
CP_LAST = N - 15 * CP_CHUNK
DEGW = 16

_MESH = dict(core_axis_name="c", subcore_axis_name="s")
_SC_PARAMS = pltpu.CompilerParams(use_tc_tiling_on_sc=False)


def _num_batches(e):
    nb = -(-e // (NW * BATCH))
    return nb + (nb % 2)


def _copy_out(acc, out_hbm, core, tid):
    @pl.when(tid < NS - 1)
    def _():
        base = tid * CP_CHUNK
        pltpu.sync_copy(acc.at[pl.ds(base, CP_CHUNK)],
                        out_hbm.at[core, pl.ds(base, CP_CHUNK)])

    @pl.when(tid == NS - 1)
    def _():
        base = (NS - 1) * CP_CHUNK
        pltpu.sync_copy(acc.at[pl.ds(base, CP_LAST)],
                        out_hbm.at[core, pl.ds(base, CP_LAST)])


def _make_deg(nb):
    @functools.partial(
        pl.kernel,
        out_type=jax.ShapeDtypeStruct((NC, N, DEGW), jnp.float32),
        mesh=plsc.VectorSubcoreMesh(**_MESH),
        compiler_params=_SC_PARAMS,
        scratch_types=[
            pltpu.VMEM((nb, BATCH), jnp.int32),
            pltpu.VMEM((BATCH, DEGW), jnp.float32),
            pltpu.VMEM_SHARED((ACC_ROWS, DEGW), jnp.float32),
        ],
    )
    def deg_kernel(col_hbm, ones_hbm, zeros_hbm, out_hbm, col_v, ones_v, acc):
        core = lax.axis_index("c")
        tid = lax.axis_index("s")
        w = core * NS + tid
        pltpu.sync_copy(col_hbm.at[w], col_v)
        pltpu.sync_copy(ones_hbm, ones_v)
        pltpu.sync_copy(zeros_hbm, acc.at[pl.ds(tid * ZROWS, ZROWS)])
        plsc.subcore_barrier()

        @pl.loop(0, nb)
        def _(j):
            pltpu.sync_copy(ones_v, acc.at[col_v.at[j]], add=True)

        plsc.subcore_barrier()
        _copy_out(acc, out_hbm, core, tid)

    return deg_kernel


def _make_spmm(npass, nb):
    shp = jax.ShapeDtypeStruct((NC, N, DH), jnp.float32)

    @functools.partial(
        pl.kernel,
        out_type=[shp] * npass,
        mesh=plsc.VectorSubcoreMesh(**_MESH),
        compiler_params=_SC_PARAMS,
        scratch_types=[
            pltpu.VMEM((nb, BATCH), jnp.int32),
            pltpu.VMEM((nb, BATCH), jnp.int32),
            pltpu.VMEM((BATCH, DH), jnp.float32),
            pltpu.VMEM((BATCH, DH), jnp.float32),
            pltpu.VMEM((BATCH, DH), jnp.float32),
            pltpu.VMEM((BATCH, DH), jnp.float32),
            pltpu.VMEM_SHARED((ACC_ROWS, DH), jnp.float32),
            pltpu.SemaphoreType.DMA,
            pltpu.SemaphoreType.DMA,
            pltpu.SemaphoreType.DMA,
            pltpu.SemaphoreType.DMA,
        ],
    )
    def spmm_kernel(row_hbm, col_hbm, *rest):
        hs = rest[:npass]
        zeros_hbm = rest[npass]
        outs = rest[npass + 1:npass + 1 + npass]
        (row_v, col_v, g0, g1, g2, g3, acc,
         sem0, sem1, sem2, sem3) = rest[npass + 1 + npass:]
        core = lax.axis_index("c")
        tid = lax.axis_index("s")
        w = core * NS + tid
        pltpu.sync_copy(row_hbm.at[w], row_v)
        pltpu.sync_copy(col_hbm.at[w], col_v)

        nbuf = 4
        bufs = (g0, g1, g2, g3)
        sems = (sem0, sem1, sem2, sem3)
        for b in range(nbuf):
            pltpu.async_copy(hs[0].at[col_v.at[b]], bufs[b], sems[b])
        for p in range(npass):
            h_hbm = hs[p]
            pltpu.sync_copy(zeros_hbm, acc.at[pl.ds(tid * ZROWS, ZROWS)])
            plsc.subcore_barrier()

            @pl.loop(0, nb, step=nbuf)
            def _(j0):
                for b in range(nbuf):
                    j = j0 + b
                    pltpu.make_async_copy(
                        h_hbm.at[col_v.at[j]], bufs[b], sems[b]).wait()
                    pltpu.sync_copy(bufs[b], acc.at[row_v.at[j]], add=True)

                    @pl.when(j + nbuf < nb)
                    def _():
                        pltpu.async_copy(
                            h_hbm.at[col_v.at[j + nbuf]], bufs[b], sems[b])

            if p + 1 < npass:
                for b in range(nbuf):
                    pltpu.async_copy(
                        hs[p + 1].at[col_v.at[b]], bufs[b], sems[b])
            plsc.subcore_barrier()
            _copy_out(acc, outs[p], core, tid)
            plsc.subcore_barrier()

    return spmm_kernel


BM = 5000


def _half_specs(n_arr, i3=False):
    if i3:
        return [pl.BlockSpec((NC, BM, DH), lambda i: (0, i, 0))] * n_arr
    return [pl.BlockSpec((BM, DH), lambda i: (i, 0))] * n_arr


def _tc_matmul_body(x_ref, w_ref, t_ref):
    t_ref[...] = jnp.dot(x_ref[...], w_ref[...],
                         preferred_element_type=jnp.float32)


def _tc_matmul(x, w1):
    return pl.pallas_call(
        _tc_matmul_body,
        grid=(N // BM,),
        in_specs=[
            pl.BlockSpec((BM, NFEAT), lambda i: (i, 0)),
            pl.BlockSpec((NFEAT, NHID), lambda i: (0, 0)),
        ],
        out_specs=pl.BlockSpec((BM, NHID), lambda i: (i, 0)),
        out_shape=jax.ShapeDtypeStruct((N, NHID), jnp.float32),
    )(x, w1)


def _tc_first_body(degp_ref, t_ref, tlo_ref, thi_ref, dinv_ref):
    deg = degp_ref[0, :, 0:1] + degp_ref[1, :, 0:1] + 1.0
    dinv = lax.rsqrt(deg)
    tp = dinv * t_ref[...]
    tlo_ref[...] = tp[:, :DH]
    thi_ref[...] = tp[:, DH:]
    dinv_ref[...] = jnp.broadcast_to(dinv, (BM, NFEAT))


def _tc_first(degp, t1, w1):
    del w1
    return pl.pallas_call(
        _tc_first_body,
        grid=(N // BM,),
        in_specs=[
            pl.BlockSpec((NC, BM, DEGW), lambda i: (0, i, 0)),
            pl.BlockSpec((BM, NFEAT), lambda i: (i, 0)),
        ],
        out_specs=_half_specs(2) + [
            pl.BlockSpec((BM, NFEAT), lambda i: (i, 0))],
        out_shape=[
            jax.ShapeDtypeStruct((N, DH), jnp.float32),
            jax.ShapeDtypeStruct((N, DH), jnp.float32),
            jax.ShapeDtypeStruct((N, NFEAT), jnp.float32),
        ],
    )(degp, t1)


def _tc_mid_body(slo_ref, shi_ref, tlo_ref, thi_ref, dinv_ref, b_ref, w_ref,
                 *out_refs, dout):
    s = jnp.concatenate(
        [slo_ref[0] + slo_ref[1] + tlo_ref[...],
         shi_ref[0] + shi_ref[1] + thi_ref[...]], axis=1)
    h = jnp.maximum(dinv_ref[...] * s + b_ref[...], 0.0)
    t = jnp.dot(h, w_ref[...], preferred_element_type=jnp.float32)
    tp = dinv_ref[:, :dout] * t
    if len(out_refs) == 2:
        out_refs[0][...] = tp[:, :DH]
        out_refs[1][...] = tp[:, DH:]
    else:
        out_refs[0][...] = tp


def _tc_mid(slo, shi, tlo, thi, dinv, b, w):
    din, dout = w.shape
    n_out = dout // DH
    out_sds = jax.ShapeDtypeStruct((N, DH), jnp.float32)
    return pl.pallas_call(
        functools.partial(_tc_mid_body, dout=dout),
        grid=(N // BM,),
        in_specs=_half_specs(2, i3=True) + _half_specs(2) + [
            pl.BlockSpec((BM, NFEAT), lambda i: (i, 0)),
            pl.BlockSpec((1, din), lambda i: (0, 0)),
            pl.BlockSpec((din, dout), lambda i: (0, 0)),
        ],
        out_specs=_half_specs(n_out),
        out_shape=[out_sds] * n_out,
    )(slo, shi, tlo, thi, dinv, b, w)


def _tc_last_body(sp_ref, tp_ref, dinv_ref, b_ref, out_ref):
    z = dinv_ref[:, :NCLASS] * (sp_ref[0] + sp_ref[1] + tp_ref[...])
    z = z + b_ref[...]
    m = jnp.max(z, axis=1, keepdims=True)
    lse = jnp.log(jnp.sum(jnp.exp(z - m), axis=1, keepdims=True)) + m
    out_ref[...] = z - lse


def _tc_last(sp, tp, dinv, b):
    return pl.pallas_call(
        _tc_last_body,
        grid=(N // BM,),
        in_specs=_half_specs(1, i3=True) + _half_specs(1) + [
            pl.BlockSpec((BM, NFEAT), lambda i: (i, 0)),
            pl.BlockSpec((1, NCLASS), lambda i: (0, 0)),
        ],
        out_specs=pl.BlockSpec((BM, NCLASS), lambda i: (i, 0)),
        out_shape=jax.ShapeDtypeStruct((N, NCLASS), jnp.float32),
    )(sp, tp, dinv, b)


def kernel(x, adj, W1, b1, W2, b2, W3, b3):
    row, col = adj[0], adj[1]
    e = row.shape[0]
    nb = _num_batches(e)
    epad = NW * nb * BATCH
    pad = epad - e
    shape3 = (NW, nb, BATCH)
    trash = N + (jnp.arange(pad, dtype=jnp.int32) % (ACC_ROWS - N))
    spread = jnp.arange(pad, dtype=jnp.int32) % N
    row_p = jnp.concatenate([row, trash]).reshape(shape3)
    col_g = jnp.concatenate([col, spread]).reshape(shape3)
    col_d = jnp.concatenate([col, trash]).reshape(shape3)

    ones_src = jnp.ones((BATCH, DEGW), jnp.float32)
    zer_deg = jnp.zeros((ZROWS, DEGW), jnp.float32)
    zer_h = jnp.zeros((ZROWS, DH), jnp.float32)

    spmm2 = _make_spmm(2, nb)
    spmm1 = _make_spmm(1, nb)

    t1 = _tc_matmul(x, W1)
    degp = _make_deg(nb)(col_d, ones_src, zer_deg)
    t1lo, t1hi, dinv = _tc_first(degp, t1, W1)
    s1lo, s1hi = spmm2(row_p, col_g, t1lo, t1hi, zer_h)
    t2lo, t2hi = _tc_mid(s1lo, s1hi, t1lo, t1hi, dinv,
                         b1.reshape(1, NHID), W2)
    s2lo, s2hi = spmm2(row_p, col_g, t2lo, t2hi, zer_h)
    (t3p,) = _tc_mid(s2lo, s2hi, t2lo, t2hi, dinv, b2.reshape(1, NHID), W3)
    (s3,) = spmm1(row_p, col_g, t3p, zer_h)
    return _tc_last(s3, t3p, dinv, b3.reshape(1, NCLASS))


# speedup vs baseline: 1.0070x; 1.0070x over previous
import functools

import jax
import jax.numpy as jnp
from jax import lax
from jax.experimental import pallas as pl
from jax.experimental.pallas import tpu as pltpu
from jax.experimental.pallas import tpu_sc as plsc

N = 10000
NFEAT = 128
NHID = 128
NCLASS = 64
DH = 64

_INFO = plsc.get_sparse_core_info()
NC = _INFO.num_cores
NS = _INFO.num_subcores
NW = NC * NS
BATCH = 128
ZROWS = 632
ACC_ROWS = NS * ZROWS
CP_CHUNK = 640
CP_LAST = N - 15 * CP_CHUNK
DEGW = 16

_MESH = dict(core_axis_name="c", subcore_axis_name="s")
_SC_PARAMS = pltpu.CompilerParams(use_tc_tiling_on_sc=False)


def _num_batches(e):
    nb = -(-e // (NW * BATCH))
    return nb + (nb % 2)


def _copy_out(acc, out_hbm, core, tid):
    @pl.when(tid < NS - 1)
    def _():
        base = tid * CP_CHUNK
        pltpu.sync_copy(acc.at[pl.ds(base, CP_CHUNK)],
                        out_hbm.at[core, pl.ds(base, CP_CHUNK)])

    @pl.when(tid == NS - 1)
    def _():
        base = (NS - 1) * CP_CHUNK
        pltpu.sync_copy(acc.at[pl.ds(base, CP_LAST)],
                        out_hbm.at[core, pl.ds(base, CP_LAST)])


def _make_deg(nb):
    @functools.partial(
        pl.kernel,
        out_type=jax.ShapeDtypeStruct((NC, N, DEGW), jnp.float32),
        mesh=plsc.VectorSubcoreMesh(**_MESH),
        compiler_params=_SC_PARAMS,
        scratch_types=[
            pltpu.VMEM((nb, BATCH), jnp.int32),
            pltpu.VMEM((BATCH, DEGW), jnp.float32),
            pltpu.VMEM_SHARED((ACC_ROWS, DEGW), jnp.float32),
        ],
    )
    def deg_kernel(col_hbm, ones_hbm, zeros_hbm, out_hbm, col_v, ones_v, acc):
        core = lax.axis_index("c")
        tid = lax.axis_index("s")
        w = core * NS + tid
        pltpu.sync_copy(col_hbm.at[w], col_v)
        pltpu.sync_copy(ones_hbm, ones_v)
        pltpu.sync_copy(zeros_hbm, acc.at[pl.ds(tid * ZROWS, ZROWS)])
        plsc.subcore_barrier()

        @pl.loop(0, nb)
        def _(j):
            pltpu.sync_copy(ones_v, acc.at[col_v.at[j]], add=True)

        plsc.subcore_barrier()
        _copy_out(acc, out_hbm, core, tid)

    return deg_kernel


def _make_spmm(npass, nb):
    shp = jax.ShapeDtypeStruct((NC, N, DH), jnp.float32)

    @functools.partial(
        pl.kernel,
        out_type=[shp] * npass,
        mesh=plsc.VectorSubcoreMesh(**_MESH),
        compiler_params=_SC_PARAMS,
        scratch_types=[
            pltpu.VMEM((nb, BATCH), jnp.int32),
            pltpu.VMEM((nb, BATCH), jnp.int32),
            pltpu.VMEM((BATCH, DH), jnp.float32),
            pltpu.VMEM((BATCH, DH), jnp.float32),
            pltpu.VMEM((BATCH, DH), jnp.float32),
            pltpu.VMEM((BATCH, DH), jnp.float32),
            pltpu.VMEM_SHARED((ACC_ROWS, DH), jnp.float32),
            pltpu.SemaphoreType.DMA,
            pltpu.SemaphoreType.DMA,
            pltpu.SemaphoreType.DMA,
            pltpu.SemaphoreType.DMA,
        ],
    )
    def spmm_kernel(row_hbm, col_hbm, *rest):
        hs = rest[:npass]
        zeros_hbm = rest[npass]
        outs = rest[npass + 1:npass + 1 + npass]
        (row_v, col_v, g0, g1, g2, g3, acc,
         sem0, sem1, sem2, sem3) = rest[npass + 1 + npass:]
        core = lax.axis_index("c")
        tid = lax.axis_index("s")
        w = core * NS + tid
        pltpu.sync_copy(row_hbm.at[w], row_v)
        pltpu.sync_copy(col_hbm.at[w], col_v)

        nbuf = 4
        bufs = (g0, g1, g2, g3)
        sems = (sem0, sem1, sem2, sem3)
        for b in range(nbuf):
            pltpu.async_copy(hs[0].at[col_v.at[b]], bufs[b], sems[b])
        for p in range(npass):
            h_hbm = hs[p]
            pltpu.sync_copy(zeros_hbm, acc.at[pl.ds(tid * ZROWS, ZROWS)])
            plsc.subcore_barrier()

            @pl.loop(0, nb, step=nbuf)
            def _(j0):
                for b in range(nbuf):
                    j = j0 + b
                    pltpu.make_async_copy(
                        h_hbm.at[col_v.at[j]], bufs[b], sems[b]).wait()
                    pltpu.sync_copy(bufs[b], acc.at[row_v.at[j]], add=True)

                    @pl.when(j + nbuf < nb)
                    def _():
                        pltpu.async_copy(
                            h_hbm.at[col_v.at[j + nbuf]], bufs[b], sems[b])

            if p + 1 < npass:
                for b in range(nbuf):
                    pltpu.async_copy(
                        hs[p + 1].at[col_v.at[b]], bufs[b], sems[b])
            plsc.subcore_barrier()
            _copy_out(acc, outs[p], core, tid)
            plsc.subcore_barrier()

    return spmm_kernel


BM = 5000


def _half_specs(n_arr, i3=False):
    if i3:
        return [pl.BlockSpec((NC, BM, DH), lambda i: (0, i, 0))] * n_arr
    return [pl.BlockSpec((BM, DH), lambda i: (i, 0))] * n_arr


def _tc_first_body(degp_ref, x_ref, w_ref, tlo_ref, thi_ref, dinv_ref):
    deg = degp_ref[0, :, 0:1] + degp_ref[1, :, 0:1] + 1.0
    dinv = lax.rsqrt(deg)
    t = jnp.dot(x_ref[...], w_ref[...], preferred_element_type=jnp.float32)
    tp = dinv * t
    tlo_ref[...] = tp[:, :DH]
    thi_ref[...] = tp[:, DH:]
    dinv_ref[...] = jnp.broadcast_to(dinv, (BM, NFEAT))


def _tc_first(degp, x, w1):
    return pl.pallas_call(
        _tc_first_body,
        grid=(N // BM,),
        in_specs=[
            pl.BlockSpec((NC, BM, DEGW), lambda i: (0, i, 0)),
            pl.BlockSpec((BM, NFEAT), lambda i: (i, 0)),
            pl.BlockSpec((NFEAT, NHID), lambda i: (0, 0)),
        ],
        out_specs=_half_specs(2) + [
            pl.BlockSpec((BM, NFEAT), lambda i: (i, 0))],
        out_shape=[
            jax.ShapeDtypeStruct((N, DH), jnp.float32),
            jax.ShapeDtypeStruct((N, DH), jnp.float32),
            jax.ShapeDtypeStruct((N, NFEAT), jnp.float32),
        ],
    )(degp, x, w1)


def _tc_mid_body(slo_ref, shi_ref, tlo_ref, thi_ref, dinv_ref, b_ref, w_ref,
                 *out_refs, dout):
    s = jnp.concatenate(
        [slo_ref[0] + slo_ref[1] + tlo_ref[...],
         shi_ref[0] + shi_ref[1] + thi_ref[...]], axis=1)
    h = jnp.maximum(dinv_ref[...] * s + b_ref[...], 0.0)
    t = jnp.dot(h, w_ref[...], preferred_element_type=jnp.float32)
    tp = dinv_ref[:, :dout] * t
    if len(out_refs) == 2:
        out_refs[0][...] = tp[:, :DH]
        out_refs[1][...] = tp[:, DH:]
    else:
        out_refs[0][...] = tp


def _tc_mid(slo, shi, tlo, thi, dinv, b, w):
    din, dout = w.shape
    n_out = dout // DH
    out_sds = jax.ShapeDtypeStruct((N, DH), jnp.float32)
    return pl.pallas_call(
        functools.partial(_tc_mid_body, dout=dout),
        grid=(N // BM,),
        in_specs=_half_specs(2, i3=True) + _half_specs(2) + [
            pl.BlockSpec((BM, NFEAT), lambda i: (i, 0)),
            pl.BlockSpec((1, din), lambda i: (0, 0)),
            pl.BlockSpec((din, dout), lambda i: (0, 0)),
        ],
        out_specs=_half_specs(n_out),
        out_shape=[out_sds] * n_out,
    )(slo, shi, tlo, thi, dinv, b, w)


def _tc_last_body(sp_ref, tp_ref, dinv_ref, b_ref, out_ref):
    z = dinv_ref[:, :NCLASS] * (sp_ref[0] + sp_ref[1] + tp_ref[...])
    z = z + b_ref[...]
    m = jnp.max(z, axis=1, keepdims=True)
    lse = jnp.log(jnp.sum(jnp.exp(z - m), axis=1, keepdims=True)) + m
    out_ref[...] = z - lse


def _tc_last(sp, tp, dinv, b):
    return pl.pallas_call(
        _tc_last_body,
        grid=(N // BM,),
        in_specs=_half_specs(1, i3=True) + _half_specs(1) + [
            pl.BlockSpec((BM, NFEAT), lambda i: (i, 0)),
            pl.BlockSpec((1, NCLASS), lambda i: (0, 0)),
        ],
        out_specs=pl.BlockSpec((BM, NCLASS), lambda i: (i, 0)),
        out_shape=jax.ShapeDtypeStruct((N, NCLASS), jnp.float32),
    )(sp, tp, dinv, b)


def kernel(x, adj, W1, b1, W2, b2, W3, b3):
    row, col = adj[0], adj[1]
    e = row.shape[0]
    nb = _num_batches(e)
    epad = NW * nb * BATCH
    pad = epad - e
    shape3 = (NW, nb, BATCH)
    trash = N + (jnp.arange(pad, dtype=jnp.int32) % (ACC_ROWS - N))
    spread = jnp.arange(pad, dtype=jnp.int32) % N
    row_p = jnp.concatenate([row, trash]).reshape(shape3)
    col_g = jnp.concatenate([col, spread]).reshape(shape3)
    col_d = jnp.concatenate([col, trash]).reshape(shape3)

    ones_src = jnp.ones((BATCH, DEGW), jnp.float32)
    zer_deg = jnp.zeros((ZROWS, DEGW), jnp.float32)
    zer_h = jnp.zeros((ZROWS, DH), jnp.float32)

    spmm2 = _make_spmm(2, nb)
    spmm1 = _make_spmm(1, nb)

    degp = _make_deg(nb)(col_d, ones_src, zer_deg)
    t1lo, t1hi, dinv = _tc_first(degp, x, W1)
    s1lo, s1hi = spmm2(row_p, col_g, t1lo, t1hi, zer_h)
    t2lo, t2hi = _tc_mid(s1lo, s1hi, t1lo, t1hi, dinv,
                         b1.reshape(1, NHID), W2)
    s2lo, s2hi = spmm2(row_p, col_g, t2lo, t2hi, zer_h)
    (t3p,) = _tc_mid(s2lo, s2hi, t2lo, t2hi, dinv, b2.reshape(1, NHID), W3)
    (s3,) = spmm1(row_p, col_g, t3p, zer_h)
    return _tc_last(s3, t3p, dinv, b3.reshape(1, NCLASS))
